# Initial kernel scaffold; baseline (speedup 1.0000x reference)
import jax
import jax.numpy as jnp
from jax.experimental import pallas as pl


def kernel(features, memory_bank):
    raise NotImplementedError("write your pallas kernel here")


# speedup vs baseline: 2.3420x; 2.3420x over previous
import functools

import jax
import jax.numpy as jnp
from jax.experimental import pallas as pl

_MEMORY_DIM = 128
_MEMORY_SIZE = 64
_TEMPERATURE = 0.07
_EPS = 1e-12


def _ema_block_kernel(x_ref, mb_ref, o_ref):
    x = x_ref[...]
    mb = mb_ref[...]

    mb_n = jnp.sqrt(jnp.sum(mb * mb, axis=1, keepdims=True))
    mb = mb / jnp.maximum(mb_n, _EPS)

    x_n = jnp.sqrt(jnp.sum(x * x, axis=1, keepdims=True))
    xn = x / jnp.maximum(x_n, _EPS)

    s = jnp.dot(xn, mb.T, preferred_element_type=jnp.float32) / _TEMPERATURE
    s_max = jnp.max(s, axis=1, keepdims=True)
    e = jnp.exp(s - s_max)
    a = e / jnp.sum(e, axis=1, keepdims=True)

    r = jnp.dot(a, mb, preferred_element_type=jnp.float32)
    o_ref[...] = xn + r


@functools.partial(jax.jit, static_argnames=("block_rows",))
def _ema_forward(features, memory_bank, block_rows=4096):
    batch, seq, dim = features.shape
    flat = features.reshape(batch * seq, dim)
    n_rows = flat.shape[0]
    grid = (n_rows // block_rows,)

    out = pl.pallas_call(
        _ema_block_kernel,
        grid=grid,
        in_specs=[
            pl.BlockSpec((block_rows, dim), lambda i: (i, 0)),
            pl.BlockSpec((_MEMORY_SIZE, dim), lambda i: (0, 0)),
        ],
        out_specs=pl.BlockSpec((block_rows, dim), lambda i: (i, 0)),
        out_shape=jax.ShapeDtypeStruct((n_rows, dim), flat.dtype),
    )(flat, memory_bank)
    return out.reshape(batch, seq, dim)


def kernel(features, memory_bank):
    return _ema_forward(features, memory_bank)


# speedup vs baseline: 2.5595x; 1.0929x over previous
import functools

import jax
import jax.numpy as jnp
from jax.experimental import pallas as pl

_MEMORY_DIM = 128
_MEMORY_SIZE = 64
_TEMPERATURE = 0.07
_EPS = 1e-12


def _ema_block_kernel(x_ref, mb_ref, o_ref):
    x = x_ref[...]
    mb = mb_ref[...]

    mb_n = jnp.sqrt(jnp.sum(mb * mb, axis=1, keepdims=True))
    mb = mb / jnp.maximum(mb_n, _EPS)

    x_n = jnp.sqrt(jnp.sum(x * x, axis=1, keepdims=True))
    xn = x / jnp.maximum(x_n, _EPS)

    s = jnp.dot(xn, mb.T, preferred_element_type=jnp.float32) / _TEMPERATURE
    s_max = jnp.max(s, axis=1, keepdims=True)
    e = jnp.exp(s - s_max)
    a = e / jnp.sum(e, axis=1, keepdims=True)

    r = jnp.dot(a, mb, preferred_element_type=jnp.float32)
    o_ref[...] = xn + r


@functools.partial(jax.jit, static_argnames=("block_rows",))
def _ema_forward(features, memory_bank, block_rows=8192):
    batch, seq, dim = features.shape
    flat = features.reshape(batch * seq, dim)
    n_rows = flat.shape[0]
    grid = (n_rows // block_rows,)

    out = pl.pallas_call(
        _ema_block_kernel,
        grid=grid,
        in_specs=[
            pl.BlockSpec((block_rows, dim), lambda i: (i, 0)),
            pl.BlockSpec((_MEMORY_SIZE, dim), lambda i: (0, 0)),
        ],
        out_specs=pl.BlockSpec((block_rows, dim), lambda i: (i, 0)),
        out_shape=jax.ShapeDtypeStruct((n_rows, dim), flat.dtype),
    )(flat, memory_bank)
    return out.reshape(batch, seq, dim)


def kernel(features, memory_bank):
    return _ema_forward(features, memory_bank)
